# Initial kernel scaffold; baseline (speedup 1.0000x reference)
import jax
import jax.numpy as jnp
from jax.experimental import pallas as pl


def kernel(x, edge_index, W_in, b_in, gcn_W, gcn_b, W1, b1, W2, b2):
    raise NotImplementedError("write your pallas kernel here")


# speedup vs baseline: 7.8671x; 7.8671x over previous
import functools

import jax
import jax.numpy as jnp
from jax import lax
from jax.experimental import pallas as pl
from jax.experimental.pallas import tpu as pltpu
from jax.experimental.pallas import tpu_sc as plsc

_CORES = 2
_TILES = 16
_K = 128
_GRID = 8


def _round_up(a: int, b: int) -> int:
    return (a + b - 1) // b * b


def _mesh():
    return plsc.VectorSubcoreMesh(
        core_axis_name="c", subcore_axis_name="s",
        num_cores=_CORES, num_subcores=_TILES)


@functools.lru_cache(maxsize=None)
def _deg_kernel(n_pad: int, ept: int, nchunk: int):
    rpt = n_pad // _TILES

    @functools.partial(
        pl.kernel,
        out_type=jax.ShapeDtypeStruct((_CORES * n_pad, 16), jnp.float32),
        mesh=_mesh(),
        scratch_types=[
            pltpu.VMEM((_K,), jnp.int32),
            pltpu.VMEM((_K, 16), jnp.float32),
            pltpu.VMEM_SHARED((n_pad, 16), jnp.float32),
        ],
    )
    def deg_k(dst_hbm, ones_hbm, zeros_hbm, out_hbm, dst_v, ones_v, acc_sh):
        c = lax.axis_index("c")
        s = lax.axis_index("s")
        pltpu.sync_copy(zeros_hbm, acc_sh.at[pl.ds(s * rpt, rpt)])
        pltpu.sync_copy(ones_hbm, ones_v)
        plsc.subcore_barrier()

        def body(i, carry):
            base = c * (ept * _TILES) + s * ept + i * _K
            pltpu.sync_copy(dst_hbm.at[pl.ds(base, _K)], dst_v)
            pltpu.sync_copy(ones_v, acc_sh.at[dst_v], add=True)
            return carry

        lax.fori_loop(0, nchunk, body, 0)
        plsc.subcore_barrier()
        pltpu.sync_copy(acc_sh.at[pl.ds(s * rpt, rpt)],
                        out_hbm.at[pl.ds(c * n_pad + s * rpt, rpt)])

    return deg_k


@functools.lru_cache(maxsize=None)
def _scatter_kernel(n_pad: int, d: int, ept: int, nchunk: int):
    rpt = n_pad // _TILES

    @functools.partial(
        pl.kernel,
        out_type=jax.ShapeDtypeStruct((_CORES * n_pad, d), jnp.float32),
        mesh=_mesh(),
        scratch_types=[
            pltpu.VMEM((_K,), jnp.int32),
            pltpu.VMEM((_K,), jnp.int32),
            pltpu.VMEM((_K, d), jnp.float32),
            pltpu.VMEM_SHARED((n_pad, d), jnp.float32),
            pltpu.SemaphoreType.DMA,
        ],
    )
    def scat_k(src_hbm, dst_hbm, table_hbm, zeros_hbm, out_hbm,
               src_v, dst_v, rows_v, acc_sh, sem):
        c = lax.axis_index("c")
        s = lax.axis_index("s")
        pltpu.sync_copy(zeros_hbm, acc_sh.at[pl.ds(s * rpt, rpt)])
        plsc.subcore_barrier()

        def body(i, carry):
            base = c * (ept * _TILES) + s * ept + i * _K
            pltpu.sync_copy(src_hbm.at[pl.ds(base, _K)], src_v)
            pltpu.sync_copy(dst_hbm.at[pl.ds(base, _K)], dst_v)
            pltpu.async_copy(table_hbm.at[src_v], rows_v, sem).wait()
            pltpu.sync_copy(rows_v, acc_sh.at[dst_v], add=True)
            return carry

        lax.fori_loop(0, nchunk, body, 0)
        plsc.subcore_barrier()
        pltpu.sync_copy(acc_sh.at[pl.ds(s * rpt, rpt)],
                        out_hbm.at[pl.ds(c * n_pad + s * rpt, rpt)])

    return scat_k


def _tc0_body(n, r_blk, dega, degb, x, w_in, b_in, w0, hws_ref, dis_ref):
    i = pl.program_id(0)
    deg = dega[...] + degb[...] + 1.0
    dis = jnp.max(lax.rsqrt(deg), axis=1, keepdims=True)
    ridx = lax.broadcasted_iota(jnp.int32, (r_blk, 1), 0) + i * r_blk
    dis = jnp.where(ridx < n, dis, 0.0)
    h0 = jnp.dot(x[...], w_in[...], preferred_element_type=jnp.float32)
    h0 = h0 + b_in[...]
    hws_ref[...] = jnp.dot(h0, w0[...],
                           preferred_element_type=jnp.float32) * dis
    dis_ref[...] = dis


def _tcl_body(acc0, acc1, hws_p, dis, b, w, out_ref):
    d = dis[...]
    h = jnp.maximum((acc0[...] + acc1[...] + hws_p[...]) * d + b[...], 0.0)
    out_ref[...] = jnp.dot(h, w[...], preferred_element_type=jnp.float32) * d


def _tc6_body(n, r_blk, acc0, acc1, hws_p, dis, b, w1, b1, w2, b2,
              out_ref, acc_s):
    i = pl.program_id(0)
    h = jnp.maximum(
        (acc0[...] + acc1[...] + hws_p[...]) * dis[...] + b[...], 0.0)
    ridx = lax.broadcasted_iota(jnp.int32, (r_blk, 1), 0) + i * r_blk
    h = jnp.where(ridx < n, h, 0.0)
    psum = jnp.sum(h, axis=0, keepdims=True)

    @pl.when(i == 0)
    def _():
        acc_s[...] = psum

    @pl.when(i > 0)
    def _():
        acc_s[...] = acc_s[...] + psum

    @pl.when(i == _GRID - 1)
    def _():
        pooled = acc_s[...] * (1.0 / n)
        hid = jnp.maximum(
            jnp.dot(pooled, w1[...],
                    preferred_element_type=jnp.float32) + b1[...], 0.0)
        out_ref[...] = jnp.dot(hid, w2[...],
                               preferred_element_type=jnp.float32) + b2[...]


def kernel(x, edge_index, W_in, b_in, gcn_W, gcn_b, W1, b1, W2, b2):
    n, d = x.shape
    e = edge_index.shape[1]
    nl = gcn_W.shape[0]
    nt = W2.shape[1]
    f32 = jnp.float32

    n_pad = _round_up(n + 1, 1024)
    e_pad = _round_up(e, _CORES * _TILES * _K)
    ept = e_pad // (_CORES * _TILES)
    nchunk = ept // _K
    rpt = n_pad // _TILES
    r_blk = n_pad // _GRID

    pad_e = jnp.full((e_pad - e,), n, edge_index.dtype)
    src = jnp.concatenate([edge_index[0], pad_e])
    dst = jnp.concatenate([edge_index[1], pad_e])
    x_pad = jnp.pad(x, ((0, n_pad - n), (0, 0)))
    ones16 = jnp.ones((_K, 16), f32)
    zeros16 = jnp.zeros((rpt, 16), f32)
    zeros_d = jnp.zeros((rpt, d), f32)
    b_in2 = b_in.reshape(1, d)

    deg2 = _deg_kernel(n_pad, ept, nchunk)(dst, ones16, zeros16)

    row_spec = pl.BlockSpec((r_blk, d), lambda i: (i, 0))
    dis_spec = pl.BlockSpec((r_blk, 1), lambda i: (i, 0))
    acc0_spec = pl.BlockSpec((r_blk, d), lambda i: (i, 0))
    acc1_spec = pl.BlockSpec((r_blk, d), lambda i: (i + _GRID, 0))
    w_spec = pl.BlockSpec((d, d), lambda i: (0, 0))
    b_spec = pl.BlockSpec((1, d), lambda i: (0, 0))

    hws, dis = pl.pallas_call(
        functools.partial(_tc0_body, n, r_blk),
        grid=(_GRID,),
        in_specs=[
            pl.BlockSpec((r_blk, 16), lambda i: (i, 0)),
            pl.BlockSpec((r_blk, 16), lambda i: (i + _GRID, 0)),
            row_spec, w_spec, b_spec, w_spec,
        ],
        out_specs=[row_spec, dis_spec],
        out_shape=[
            jax.ShapeDtypeStruct((n_pad, d), f32),
            jax.ShapeDtypeStruct((n_pad, 1), f32),
        ],
    )(deg2, deg2, x_pad, W_in, b_in2, gcn_W[0])

    scat = _scatter_kernel(n_pad, d, ept, nchunk)
    tcl = pl.pallas_call(
        _tcl_body,
        grid=(_GRID,),
        in_specs=[acc0_spec, acc1_spec, row_spec, dis_spec, b_spec, w_spec],
        out_specs=row_spec,
        out_shape=jax.ShapeDtypeStruct((n_pad, d), f32),
    )

    for l in range(1, nl):
        acc = scat(src, dst, hws, zeros_d)
        hws = tcl(acc, acc, hws, dis, gcn_b[l - 1].reshape(1, d), gcn_W[l])

    acc = scat(src, dst, hws, zeros_d)

    w2p = jnp.pad(W2, ((0, 0), (0, d - nt)))
    b2p = jnp.pad(b2, (0, d - nt)).reshape(1, d)
    outp = pl.pallas_call(
        functools.partial(_tc6_body, n, r_blk),
        grid=(_GRID,),
        in_specs=[acc0_spec, acc1_spec, row_spec, dis_spec, b_spec,
                  w_spec, b_spec, w_spec, b_spec],
        out_specs=pl.BlockSpec((1, d), lambda i: (0, 0)),
        out_shape=jax.ShapeDtypeStruct((1, d), f32),
        scratch_shapes=[pltpu.VMEM((1, d), f32)],
    )(acc, acc, hws, dis, gcn_b[nl - 1].reshape(1, d),
      W1, b1.reshape(1, d), w2p, b2p)

    return outp[:, :nt]
